# Initial kernel scaffold; baseline (speedup 1.0000x reference)
import jax
import jax.numpy as jnp
from jax.experimental import pallas as pl


def kernel(x, Wr, Wg, Wu, Wd, expert_bias):
    raise NotImplementedError("write your pallas kernel here")


# speedup vs baseline: 1.1260x; 1.1260x over previous
import functools

import jax
import jax.numpy as jnp
from jax.experimental import pallas as pl
from jax.experimental.pallas import tpu as pltpu

T = 2048
D = 1024
E = 16
K = 2
FF = 512

BT = 256


def _fused_body(x_ref, wr_ref, bias_ref, wg_ref, wu_ref, wd_ref, y_ref, w_s):
    e = pl.program_id(1)

    @pl.when(e == 0)
    def _router():
        xb = x_ref[...]
        logits = jax.lax.dot_general(
            xb, wr_ref[...], (((1,), (1,)), ((), ())),
            preferred_element_type=jnp.float32)
        m = jnp.max(logits, axis=-1, keepdims=True)
        p = jnp.exp(logits - m)
        gates = p / jnp.sum(p, axis=-1, keepdims=True)
        g = gates + bias_ref[...]
        iota = jax.lax.broadcasted_iota(jnp.int32, (BT, E), 1)
        m1 = jnp.max(g, axis=-1, keepdims=True)
        i1 = jnp.min(jnp.where(g == m1, iota, E), axis=-1, keepdims=True)
        g2 = jnp.where(iota == i1, -1e30, g)
        m2 = jnp.max(g2, axis=-1, keepdims=True)
        i2 = jnp.min(jnp.where(g2 == m2, iota, E), axis=-1, keepdims=True)
        denom = m1 + m2 + 1e-20
        w = (jnp.where(iota == i1, m1 / denom, 0.0)
             + jnp.where(iota == i2, m2 / denom, 0.0))
        w_s[...] = w

    xb = x_ref[...]
    wg = wg_ref[0]
    wu = wu_ref[0]
    wd = wd_ref[0]
    hg = jax.lax.dot_general(xb, wg, (((1,), (1,)), ((), ())),
                             preferred_element_type=jnp.float32)
    hu = jax.lax.dot_general(xb, wu, (((1,), (1,)), ((), ())),
                             preferred_element_type=jnp.float32)
    h = hg * jax.lax.logistic(hg) * hu
    ye = jax.lax.dot_general(h, wd, (((1,), (1,)), ((), ())),
                             preferred_element_type=jnp.float32)
    iota_e = jax.lax.broadcasted_iota(jnp.int32, (BT, E), 1)
    w_col = jnp.sum(jnp.where(iota_e == e, w_s[...], 0.0), axis=1, keepdims=True)
    contrib = w_col * ye

    @pl.when(e == 0)
    def _init():
        y_ref[...] = contrib

    @pl.when(e > 0)
    def _acc():
        y_ref[...] += contrib


def kernel(x, Wr, Wg, Wu, Wd, expert_bias):
    bias2 = expert_bias.reshape(1, E)
    grid = (T // BT, E)
    y = pl.pallas_call(
        _fused_body,
        grid=grid,
        in_specs=[
            pl.BlockSpec((BT, D), lambda t, e: (t, 0)),
            pl.BlockSpec((E, D), lambda t, e: (0, 0)),
            pl.BlockSpec((1, E), lambda t, e: (0, 0)),
            pl.BlockSpec((1, FF, D), lambda t, e: (e, 0, 0)),
            pl.BlockSpec((1, FF, D), lambda t, e: (e, 0, 0)),
            pl.BlockSpec((1, D, FF), lambda t, e: (e, 0, 0)),
        ],
        out_specs=pl.BlockSpec((BT, D), lambda t, e: (t, 0)),
        out_shape=jax.ShapeDtypeStruct((T, D), jnp.float32),
        scratch_shapes=[pltpu.VMEM((BT, E), jnp.float32)],
    )(x, Wr, bias2, Wg, Wu, Wd)
    return y
